# Initial kernel scaffold; baseline (speedup 1.0000x reference)
import functools

import jax
import jax.numpy as jnp
from jax import lax
from jax.experimental import pallas as pl
from jax.experimental.pallas import tpu as pltpu
from jax.experimental.pallas import tpu_sc as plsc

EPS = 1e-6
NC = 2
NS = 16
NW = NC * NS
L = 16
K = 128


def _sc_mesh():
    return plsc.VectorSubcoreMesh(
        core_axis_name="c", subcore_axis_name="s", num_cores=NC, num_subcores=NS
    )


def _make_degrees(E, N):
    assert E % NW == 0 and N % L == 0
    epw = E // NW
    n_full, rem = divmod(epw, K)
    assert rem % L == 0

    @functools.partial(
        pl.kernel,
        out_type=jax.ShapeDtypeStruct((NW, 2, N), jnp.float32),
        mesh=_sc_mesh(),
        scratch_types=[
            pltpu.VMEM((N,), jnp.float32),
            pltpu.VMEM((N,), jnp.float32),
            pltpu.VMEM((K,), jnp.int32),
            pltpu.VMEM((K,), jnp.int32),
        ],
    )
    def deg_kernel(src_hbm, dst_hbm, out_hbm, hs, hd, si, di):
        c = lax.axis_index("c")
        s = lax.axis_index("s")
        wid = c * NS + s
        base = wid * epw
        zeros16 = jnp.zeros((L,), jnp.float32)
        ones16 = jnp.ones((L,), jnp.float32)

        def zero_body(i, carry):
            hs[pl.ds(i * L, L)] = zeros16
            hd[pl.ds(i * L, L)] = zeros16
            return carry

        lax.fori_loop(0, N // L, zero_body, 0)

        def chunk_body(i, carry):
            off = base + i * K
            pltpu.sync_copy(src_hbm.at[pl.ds(off, K)], si)
            pltpu.sync_copy(dst_hbm.at[pl.ds(off, K)], di)
            for j in range(K // L):
                plsc.addupdate_scatter(hs, [si[pl.ds(j * L, L)]], ones16)
                plsc.addupdate_scatter(hd, [di[pl.ds(j * L, L)]], ones16)
            return carry

        lax.fori_loop(0, n_full, chunk_body, 0)

        if rem:
            off = base + n_full * K
            pltpu.sync_copy(src_hbm.at[pl.ds(off, rem)], si.at[pl.ds(0, rem)])
            pltpu.sync_copy(dst_hbm.at[pl.ds(off, rem)], di.at[pl.ds(0, rem)])
            for j in range(rem // L):
                plsc.addupdate_scatter(hs, [si[pl.ds(j * L, L)]], ones16)
                plsc.addupdate_scatter(hd, [di[pl.ds(j * L, L)]], ones16)

        pltpu.sync_copy(hs, out_hbm.at[wid, 0])
        pltpu.sync_copy(hd, out_hbm.at[wid, 1])

    return deg_kernel


def _make_aggregate(E, N, D):
    assert E % NW == 0 and N % NS == 0
    epw = E // NW
    n_full, rem = divmod(epw, K)
    rpt = N // NS

    @functools.partial(
        pl.kernel,
        out_type=jax.ShapeDtypeStruct((NC, N, D), jnp.float32),
        mesh=_sc_mesh(),
        scratch_types=[
            pltpu.VMEM_SHARED((N, D), jnp.float32),
            pltpu.VMEM((K,), jnp.int32),
            pltpu.VMEM((K,), jnp.int32),
            pltpu.VMEM((K, D), jnp.float32),
            pltpu.VMEM((L,), jnp.int32),
            pltpu.VMEM((L,), jnp.int32),
            pltpu.VMEM((L, D), jnp.float32),
            pltpu.SemaphoreType.DMA,
        ],
    )
    def agg_kernel(h_hbm, src_hbm, dst_hbm, zeros_hbm, out_hbm,
                   acc, si, di, rows, sit, dit, rowst, sem):
        c = lax.axis_index("c")
        s = lax.axis_index("s")
        wid = c * NS + s
        base = wid * epw

        pltpu.sync_copy(zeros_hbm.at[pl.ds(s * rpt, rpt)],
                        acc.at[pl.ds(s * rpt, rpt)])
        plsc.subcore_barrier()

        def chunk_body(i, carry):
            off = base + i * K
            pltpu.sync_copy(src_hbm.at[pl.ds(off, K)], si)
            pltpu.sync_copy(dst_hbm.at[pl.ds(off, K)], di)
            pltpu.async_copy(h_hbm.at[si], rows, sem).wait()
            pltpu.sync_copy(rows, acc.at[di], add=True)
            return carry

        lax.fori_loop(0, n_full, chunk_body, 0)

        if rem:
            assert rem == L, "tail sized for one vector register chunk"
            off = base + n_full * K
            pltpu.sync_copy(src_hbm.at[pl.ds(off, rem)], sit)
            pltpu.sync_copy(dst_hbm.at[pl.ds(off, rem)], dit)
            pltpu.async_copy(h_hbm.at[sit], rowst, sem).wait()
            pltpu.sync_copy(rowst, acc.at[dit], add=True)

        plsc.subcore_barrier()
        pltpu.sync_copy(acc.at[pl.ds(s * rpt, rpt)],
                        out_hbm.at[c, pl.ds(s * rpt, rpt)])

    return agg_kernel


def _prep(x, hist_t, a2, b2, block_n):
    N, D = x.shape

    def body(x_ref, hist_ref, a2_ref, b2_ref, h_ref):
        xb = x_ref[...]
        mean = jnp.mean(xb, axis=1, keepdims=True)
        xc = xb - mean
        std = jnp.sqrt(jnp.sum(xc * xc, axis=1, keepdims=True) / (D - 1))
        hn = a2_ref[...] * xc / (std + EPS) + b2_ref[...]
        out_deg = jnp.maximum(jnp.sum(hist_ref[...][0], axis=1), 1.0)
        h_ref[...] = hn * lax.rsqrt(out_deg)[:, None]

    return pl.pallas_call(
        body,
        grid=(N // block_n,),
        in_specs=[
            pl.BlockSpec((block_n, D), lambda i: (i, 0)),
            pl.BlockSpec((2, block_n, NW), lambda i: (0, i, 0)),
            pl.BlockSpec((1, D), lambda i: (0, 0)),
            pl.BlockSpec((1, D), lambda i: (0, 0)),
        ],
        out_specs=pl.BlockSpec((block_n, D), lambda i: (i, 0)),
        out_shape=jax.ShapeDtypeStruct((N, D), jnp.float32),
    )(x, hist_t, a2.reshape(1, D), b2.reshape(1, D))


def _finish(parts, hist_t, x, W, b, block_n):
    N, D = x.shape

    def body(parts_ref, hist_ref, x_ref, w_ref, b_ref, out_ref):
        agg = parts_ref[0] + parts_ref[1]
        in_deg = jnp.maximum(jnp.sum(hist_ref[...][1], axis=1), 1.0)
        agg = agg * lax.rsqrt(in_deg)[:, None]
        out = jnp.dot(agg, w_ref[...], preferred_element_type=jnp.float32)
        out_ref[...] = jnp.maximum(out + b_ref[...], 0.0) + x_ref[...]

    return pl.pallas_call(
        body,
        grid=(N // block_n,),
        in_specs=[
            pl.BlockSpec((NC, block_n, D), lambda i: (0, i, 0)),
            pl.BlockSpec((2, block_n, NW), lambda i: (0, i, 0)),
            pl.BlockSpec((block_n, D), lambda i: (i, 0)),
            pl.BlockSpec((D, D), lambda i: (0, 0)),
            pl.BlockSpec((1, D), lambda i: (0, 0)),
        ],
        out_specs=pl.BlockSpec((block_n, D), lambda i: (i, 0)),
        out_shape=jax.ShapeDtypeStruct((N, D), jnp.float32),
    )(parts, hist_t, x, W, b.reshape(1, D))


def kernel(x, edge_index, W, b, a2, b2):
    N, D = x.shape
    E = edge_index.shape[1]
    src = edge_index[0]
    dst = edge_index[1]

    hist = _make_degrees(E, N)(src, dst)
    hist_t = jnp.transpose(hist, (1, 2, 0))

    block_n = 1000 if N % 1000 == 0 else 8
    h = _prep(x, hist_t, a2, b2, block_n)

    zeros = jnp.zeros((N, D), jnp.float32)
    parts = _make_aggregate(E, N, D)(h, src, dst, zeros)

    return _finish(parts, hist_t, x, W, b, block_n)


# speedup vs baseline: 5.8015x; 5.8015x over previous
import functools

import jax
import jax.numpy as jnp
from jax import lax
from jax.experimental import pallas as pl
from jax.experimental.pallas import tpu as pltpu
from jax.experimental.pallas import tpu_sc as plsc

EPS = 1e-6
NC = 2
NS = 16
NW = NC * NS
L = 16
K = 128


def _sc_mesh():
    return plsc.VectorSubcoreMesh(
        core_axis_name="c", subcore_axis_name="s", num_cores=NC, num_subcores=NS
    )


def _make_degrees(E, N):
    assert E % NW == 0 and N % L == 0
    epw = E // NW
    n_full, rem = divmod(epw, K)
    assert rem % L == 0

    @functools.partial(
        pl.kernel,
        out_type=jax.ShapeDtypeStruct((NW * 2 * N,), jnp.float32),
        mesh=_sc_mesh(),
        compiler_params=pltpu.CompilerParams(needs_layout_passes=False),
        scratch_types=[
            pltpu.VMEM((N,), jnp.float32),
            pltpu.VMEM((N,), jnp.float32),
            pltpu.VMEM((K,), jnp.int32),
            pltpu.VMEM((K,), jnp.int32),
        ],
    )
    def deg_kernel(src_hbm, dst_hbm, out_hbm, hs, hd, si, di):
        c = lax.axis_index("c")
        s = lax.axis_index("s")
        wid = c * NS + s
        base = wid * epw
        zeros16 = jnp.zeros((L,), jnp.float32)
        ones16 = jnp.ones((L,), jnp.float32)

        def zero_body(i, carry):
            hs[pl.ds(i * L, L)] = zeros16
            hd[pl.ds(i * L, L)] = zeros16
            return carry

        lax.fori_loop(0, N // L, zero_body, 0)

        def chunk_body(i, carry):
            off = base + i * K
            pltpu.sync_copy(src_hbm.at[pl.ds(off, K)], si)
            pltpu.sync_copy(dst_hbm.at[pl.ds(off, K)], di)
            for j in range(K // L):
                plsc.addupdate_scatter(hs, [si[pl.ds(j * L, L)]], ones16)
                plsc.addupdate_scatter(hd, [di[pl.ds(j * L, L)]], ones16)
            return carry

        lax.fori_loop(0, n_full, chunk_body, 0)

        if rem:
            off = base + n_full * K
            pltpu.sync_copy(src_hbm.at[pl.ds(off, rem)], si.at[pl.ds(0, rem)])
            pltpu.sync_copy(dst_hbm.at[pl.ds(off, rem)], di.at[pl.ds(0, rem)])
            for j in range(rem // L):
                plsc.addupdate_scatter(hs, [si[pl.ds(j * L, L)]], ones16)
                plsc.addupdate_scatter(hd, [di[pl.ds(j * L, L)]], ones16)

        pltpu.sync_copy(hs, out_hbm.at[pl.ds(wid * 2 * N, N)])
        pltpu.sync_copy(hd, out_hbm.at[pl.ds(wid * 2 * N + N, N)])

    return deg_kernel


def _make_aggregate(E, N, D):
    assert E % NW == 0 and N % NS == 0
    epw = E // NW
    n_full, rem = divmod(epw, K)
    rpt = (N // NS) // 8 * 8
    rtail = N - NS * rpt

    @functools.partial(
        pl.kernel,
        out_type=jax.ShapeDtypeStruct((NC, N, D), jnp.float32),
        mesh=_sc_mesh(),
        compiler_params=pltpu.CompilerParams(needs_layout_passes=False),
        scratch_types=[
            pltpu.VMEM_SHARED((N, D), jnp.float32),
            pltpu.VMEM((K,), jnp.int32),
            pltpu.VMEM((K,), jnp.int32),
            pltpu.VMEM((K, D), jnp.float32),
            pltpu.VMEM((L,), jnp.int32),
            pltpu.VMEM((L,), jnp.int32),
            pltpu.VMEM((L, D), jnp.float32),
            pltpu.SemaphoreType.DMA,
        ],
    )
    def agg_kernel(h_hbm, src_hbm, dst_hbm, zeros_hbm, out_hbm,
                   acc, si, di, rows, sit, dit, rowst, sem):
        c = lax.axis_index("c")
        s = lax.axis_index("s")
        wid = c * NS + s
        base = wid * epw

        pltpu.sync_copy(zeros_hbm.at[pl.ds(s * rpt, rpt)],
                        acc.at[pl.ds(s * rpt, rpt)])
        if rtail:
            @pl.when(s == NS - 1)
            def _():
                pltpu.sync_copy(zeros_hbm.at[pl.ds(NS * rpt, rtail)],
                                acc.at[pl.ds(NS * rpt, rtail)])
        plsc.subcore_barrier()

        def chunk_body(i, carry):
            off = base + i * K
            pltpu.sync_copy(src_hbm.at[pl.ds(off, K)], si)
            pltpu.sync_copy(dst_hbm.at[pl.ds(off, K)], di)
            pltpu.async_copy(h_hbm.at[si], rows, sem).wait()
            pltpu.sync_copy(rows, acc.at[di], add=True)
            return carry

        lax.fori_loop(0, n_full, chunk_body, 0)

        if rem:
            assert rem == L, "tail sized for one vector register chunk"
            off = base + n_full * K
            pltpu.sync_copy(src_hbm.at[pl.ds(off, rem)], sit)
            pltpu.sync_copy(dst_hbm.at[pl.ds(off, rem)], dit)
            pltpu.async_copy(h_hbm.at[sit], rowst, sem).wait()
            pltpu.sync_copy(rowst, acc.at[dit], add=True)

        plsc.subcore_barrier()
        pltpu.sync_copy(acc.at[pl.ds(s * rpt, rpt)],
                        out_hbm.at[c, pl.ds(s * rpt, rpt)])
        if rtail:
            @pl.when(s == NS - 1)
            def _():
                pltpu.sync_copy(acc.at[pl.ds(NS * rpt, rtail)],
                                out_hbm.at[c, pl.ds(NS * rpt, rtail)])

    return agg_kernel


def _prep(x, hist_t, a2, b2, block_n):
    N, D = x.shape

    def body(x_ref, hist_ref, a2_ref, b2_ref, h_ref):
        xb = x_ref[...]
        mean = jnp.mean(xb, axis=1, keepdims=True)
        xc = xb - mean
        std = jnp.sqrt(jnp.sum(xc * xc, axis=1, keepdims=True) / (D - 1))
        hn = a2_ref[...] * xc / (std + EPS) + b2_ref[...]
        out_deg = jnp.maximum(jnp.sum(hist_ref[...][0], axis=1), 1.0)
        h_ref[...] = hn * lax.rsqrt(out_deg)[:, None]

    return pl.pallas_call(
        body,
        grid=(N // block_n,),
        in_specs=[
            pl.BlockSpec((block_n, D), lambda i: (i, 0)),
            pl.BlockSpec((2, block_n, NW), lambda i: (0, i, 0)),
            pl.BlockSpec((1, D), lambda i: (0, 0)),
            pl.BlockSpec((1, D), lambda i: (0, 0)),
        ],
        out_specs=pl.BlockSpec((block_n, D), lambda i: (i, 0)),
        out_shape=jax.ShapeDtypeStruct((N, D), jnp.float32),
    )(x, hist_t, a2.reshape(1, D), b2.reshape(1, D))


def _finish(parts, hist_t, x, W, b, block_n):
    N, D = x.shape

    def body(parts_ref, hist_ref, x_ref, w_ref, b_ref, out_ref):
        agg = parts_ref[0] + parts_ref[1]
        in_deg = jnp.maximum(jnp.sum(hist_ref[...][1], axis=1), 1.0)
        agg = agg * lax.rsqrt(in_deg)[:, None]
        out = jnp.dot(agg, w_ref[...], preferred_element_type=jnp.float32)
        out_ref[...] = jnp.maximum(out + b_ref[...], 0.0) + x_ref[...]

    return pl.pallas_call(
        body,
        grid=(N // block_n,),
        in_specs=[
            pl.BlockSpec((NC, block_n, D), lambda i: (0, i, 0)),
            pl.BlockSpec((2, block_n, NW), lambda i: (0, i, 0)),
            pl.BlockSpec((block_n, D), lambda i: (i, 0)),
            pl.BlockSpec((D, D), lambda i: (0, 0)),
            pl.BlockSpec((1, D), lambda i: (0, 0)),
        ],
        out_specs=pl.BlockSpec((block_n, D), lambda i: (i, 0)),
        out_shape=jax.ShapeDtypeStruct((N, D), jnp.float32),
    )(parts, hist_t, x, W, b.reshape(1, D))


def kernel(x, edge_index, W, b, a2, b2):
    N, D = x.shape
    E = edge_index.shape[1]
    src = edge_index[0]
    dst = edge_index[1]

    hist = _make_degrees(E, N)(src, dst).reshape(NW, 2, N)
    hist_t = jnp.transpose(hist, (1, 2, 0))

    block_n = 1000 if N % 1000 == 0 else 8
    h = _prep(x, hist_t, a2, b2, block_n)

    zeros = jnp.zeros((N, D), jnp.float32)
    parts = _make_aggregate(E, N, D)(h, src, dst, zeros)

    return _finish(parts, hist_t, x, W, b, block_n)
